# baseline (speedup 1.0000x reference)
import functools

import jax
import jax.numpy as jnp
from jax import lax
from jax.experimental import pallas as pl
from jax.experimental.pallas import tpu as pltpu
from jax.experimental.pallas import tpu_sc as plsc

_NC = 2
_NS = 16


def _mm_body(a_ref, b_ref, o_ref):
    o_ref[...] = lax.dot_general(
        a_ref[...], b_ref[...],
        dimension_numbers=(((1,), (1,)), ((), ())),
        preferred_element_type=jnp.float32,
    )


def _project_table(A, B, block_rows):
    vocab, k = A.shape
    d = B.shape[0]
    grid = vocab // block_rows
    return pl.pallas_call(
        _mm_body,
        grid=(grid,),
        in_specs=[
            pl.BlockSpec((block_rows, k), lambda i: (i, 0)),
            pl.BlockSpec((d, k), lambda i: (0, 0)),
        ],
        out_specs=pl.BlockSpec((block_rows, d), lambda i: (i, 0)),
        out_shape=jax.ShapeDtypeStruct((vocab, d), jnp.float32),
    )(A, B)


def _make_gather(n_idx, d, per_w, chunk):
    n_chunks = per_w // chunk
    mesh = plsc.VectorSubcoreMesh(core_axis_name="c", subcore_axis_name="s")

    @functools.partial(
        pl.kernel,
        out_type=jax.ShapeDtypeStruct((n_idx, d), jnp.float32),
        mesh=mesh,
        scratch_types=[
            pltpu.VMEM((per_w,), jnp.int32),
            pltpu.VMEM((chunk, d), jnp.float32),
            pltpu.VMEM((chunk, d), jnp.float32),
            pltpu.SemaphoreType.DMA,
            pltpu.SemaphoreType.DMA,
        ],
    )
    def gather(table_hbm, idx_hbm, out_hbm, idx_v, rows0, rows1, sem0, sem1):
        wid = lax.axis_index("s") * _NC + lax.axis_index("c")
        base = wid * per_w
        pltpu.sync_copy(idx_hbm.at[pl.ds(base, per_w)], idx_v)

        pltpu.async_copy(table_hbm.at[idx_v.at[pl.ds(0, chunk)]], rows0, sem0)

        def step(i, _):
            slot = lax.rem(i, 2)

            def run(cur_rows, cur_sem, nxt_rows, nxt_sem):
                @pl.when(i + 1 < n_chunks)
                def _():
                    pltpu.async_copy(
                        table_hbm.at[idx_v.at[pl.ds((i + 1) * chunk, chunk)]],
                        nxt_rows, nxt_sem,
                    )
                pltpu.make_async_copy(
                    table_hbm.at[idx_v.at[pl.ds(i * chunk, chunk)]],
                    cur_rows, cur_sem,
                ).wait()
                pltpu.sync_copy(
                    cur_rows, out_hbm.at[pl.ds(base + i * chunk, chunk)]
                )

            @pl.when(slot == 0)
            def _():
                run(rows0, sem0, rows1, sem1)

            @pl.when(slot == 1)
            def _():
                run(rows1, sem1, rows0, sem0)

            return 0

        lax.fori_loop(0, n_chunks, step, 0)

    return gather


def _fin_body(x_ref, o_ref, *, bb, hist, d):
    o_ref[...] = x_ref[...].reshape(bb, hist, d)


def _finish(low, batch, hist, d, bb):
    rows = bb * hist
    grid = batch // bb
    body = functools.partial(_fin_body, bb=bb, hist=hist, d=d)
    return pl.pallas_call(
        body,
        grid=(grid,),
        in_specs=[pl.BlockSpec((rows, d), lambda i: (i, 0))],
        out_specs=pl.BlockSpec((bb, hist, d), lambda i: (i, 0, 0)),
        out_shape=jax.ShapeDtypeStruct((batch, hist, d), jnp.float32),
    )(low)


def kernel(token_ids, A, B):
    batch, hist = token_ids.shape
    vocab, k = A.shape
    d = B.shape[0]

    table = _project_table(A, B, block_rows=4000)

    n_idx = batch * hist
    flat_ids = token_ids.reshape(n_idx).astype(jnp.int32)

    per_w = n_idx // (_NC * _NS)
    gather = _make_gather(n_idx, d, per_w, chunk=320)
    low = gather(table, flat_ids)

    return _finish(low, batch, hist, d, bb=16)


# speedup vs baseline: 3.4013x; 3.4013x over previous
import functools

import jax
import jax.numpy as jnp
from jax import lax
from jax.experimental import pallas as pl
from jax.experimental.pallas import tpu as pltpu
from jax.experimental.pallas import tpu_sc as plsc

_NC = 2
_NS = 16


def _mm_body(at_ref, bt_ref, o_ref):
    o_ref[...] = lax.dot_general(
        at_ref[...], bt_ref[...],
        dimension_numbers=(((0,), (0,)), ((), ())),
        preferred_element_type=jnp.float32,
    )


def _project_table(At, Bt, block_cols):
    k, vocab = At.shape
    d = Bt.shape[1]
    grid = (vocab + block_cols - 1) // block_cols
    return pl.pallas_call(
        _mm_body,
        grid=(grid,),
        in_specs=[
            pl.BlockSpec((k, block_cols), lambda i: (0, i)),
            pl.BlockSpec((k, d), lambda i: (0, 0)),
        ],
        out_specs=pl.BlockSpec((block_cols, d), lambda i: (i, 0)),
        out_shape=jax.ShapeDtypeStruct((vocab, d), jnp.float32),
    )(At, Bt)


def _make_gather(n_idx, d, per_w, chunk):
    n_chunks = per_w // chunk
    mesh = plsc.VectorSubcoreMesh(core_axis_name="c", subcore_axis_name="s")

    @functools.partial(
        pl.kernel,
        out_type=jax.ShapeDtypeStruct((n_idx, d), jnp.float32),
        mesh=mesh,
        scratch_types=[
            pltpu.VMEM((per_w,), jnp.int32),
            pltpu.VMEM((chunk, d), jnp.float32),
            pltpu.VMEM((chunk, d), jnp.float32),
            pltpu.SemaphoreType.DMA,
            pltpu.SemaphoreType.DMA,
        ],
    )
    def gather(table_hbm, idx_hbm, out_hbm, idx_v, rows0, rows1, sem0, sem1):
        wid = lax.axis_index("s") * _NC + lax.axis_index("c")
        base = wid * per_w
        pltpu.sync_copy(idx_hbm.at[pl.ds(base, per_w)], idx_v)

        pltpu.async_copy(table_hbm.at[idx_v.at[pl.ds(0, chunk)]], rows0, sem0)

        def step(i, _):
            slot = lax.rem(i, 2)

            def run(cur_rows, cur_sem, nxt_rows, nxt_sem):
                @pl.when(i + 1 < n_chunks)
                def _():
                    pltpu.async_copy(
                        table_hbm.at[idx_v.at[pl.ds((i + 1) * chunk, chunk)]],
                        nxt_rows, nxt_sem,
                    )
                pltpu.make_async_copy(
                    table_hbm.at[idx_v.at[pl.ds(i * chunk, chunk)]],
                    cur_rows, cur_sem,
                ).wait()
                pltpu.sync_copy(
                    cur_rows, out_hbm.at[pl.ds(base + i * chunk, chunk)]
                )

            @pl.when(slot == 0)
            def _():
                run(rows0, sem0, rows1, sem1)

            @pl.when(slot == 1)
            def _():
                run(rows1, sem1, rows0, sem0)

            return 0

        lax.fori_loop(0, n_chunks, step, 0)

    return gather


def kernel(token_ids, A, B):
    batch, hist = token_ids.shape
    vocab, k = A.shape
    d = B.shape[0]

    table = _project_table(A.T, B.T, block_cols=6400)

    n_idx = batch * hist
    idx_hm = token_ids.T.reshape(n_idx).astype(jnp.int32)

    per_w = n_idx // (_NC * _NS)
    gather = _make_gather(n_idx, d, per_w, chunk=320)
    low = gather(table, idx_hm)

    return low.reshape(hist, batch, d).transpose(1, 0, 2)


# speedup vs baseline: 3.5440x; 1.0419x over previous
import functools

import jax
import jax.numpy as jnp
from jax import lax
from jax.experimental import pallas as pl
from jax.experimental.pallas import tpu as pltpu
from jax.experimental.pallas import tpu_sc as plsc

_NC = 2
_NS = 16


def _mm_body(at_ref, bt_ref, o_ref):
    o_ref[...] = lax.dot_general(
        at_ref[...], bt_ref[...],
        dimension_numbers=(((0,), (0,)), ((), ())),
        preferred_element_type=jnp.float32,
    )


def _project_table(At, Bt, block_cols):
    k, vocab = At.shape
    d = Bt.shape[1]
    grid = (vocab + block_cols - 1) // block_cols
    return pl.pallas_call(
        _mm_body,
        grid=(grid,),
        in_specs=[
            pl.BlockSpec((k, block_cols), lambda i: (0, i)),
            pl.BlockSpec((k, d), lambda i: (0, 0)),
        ],
        out_specs=pl.BlockSpec((block_cols, d), lambda i: (i, 0)),
        out_shape=jax.ShapeDtypeStruct((vocab, d), jnp.float32),
    )(At, Bt)


def _make_gather(n_idx, d, per_w, chunk, nbuf=3):
    n_chunks = per_w // chunk
    mesh = plsc.VectorSubcoreMesh(core_axis_name="c", subcore_axis_name="s")

    @functools.partial(
        pl.kernel,
        out_type=jax.ShapeDtypeStruct((n_idx, d), jnp.float32),
        mesh=mesh,
        scratch_types=[
            pltpu.VMEM((per_w,), jnp.int32),
        ] + [pltpu.VMEM((chunk, d), jnp.float32) for _ in range(nbuf)]
          + [pltpu.SemaphoreType.DMA for _ in range(2 * nbuf)],
    )
    def gather(table_hbm, idx_hbm, out_hbm, idx_v, *bufs_sems):
        bufs = bufs_sems[:nbuf]
        gsems = bufs_sems[nbuf:2 * nbuf]
        wsems = bufs_sems[2 * nbuf:]
        wid = lax.axis_index("s") * _NC + lax.axis_index("c")
        base = wid * per_w
        pltpu.sync_copy(idx_hbm.at[pl.ds(base, per_w)], idx_v)

        def fire_gather(i, s):
            pltpu.async_copy(
                table_hbm.at[idx_v.at[pl.ds(i * chunk, chunk)]],
                bufs[s], gsems[s])

        def wait_gather(i, s):
            pltpu.make_async_copy(
                table_hbm.at[idx_v.at[pl.ds(i * chunk, chunk)]],
                bufs[s], gsems[s]).wait()

        def fire_write(i, s):
            pltpu.async_copy(
                bufs[s], out_hbm.at[pl.ds(base + i * chunk, chunk)], wsems[s])

        def wait_write(i, s):
            pltpu.make_async_copy(
                bufs[s], out_hbm.at[pl.ds(base + i * chunk, chunk)],
                wsems[s]).wait()

        fire_gather(0, 0)
        fire_gather(1, 1)

        def step(i, _):
            for s in range(nbuf):
                @pl.when(lax.rem(i, nbuf) == s)
                def _():
                    s2 = (s + 2) % nbuf
                    @pl.when(i + 2 < n_chunks)
                    def _():
                        @pl.when(i >= 1)
                        def _():
                            wait_write(i - 1, s2)
                        fire_gather(i + 2, s2)
                    wait_gather(i, s)
                    fire_write(i, s)
            return 0

        lax.fori_loop(0, n_chunks, step, 0)

        for j in range(max(0, n_chunks - 3), n_chunks):
            wait_write(j, j % nbuf)

    return gather


def kernel(token_ids, A, B):
    batch, hist = token_ids.shape
    vocab, k = A.shape
    d = B.shape[0]

    table = _project_table(A.T, B.T, block_cols=12800)

    n_idx = batch * hist
    idx_hm = token_ids.T.reshape(n_idx).astype(jnp.int32)

    per_w = n_idx // (_NC * _NS)
    gather = _make_gather(n_idx, d, per_w, chunk=320)
    low = gather(table, idx_hm)

    return low.reshape(hist, batch, d).transpose(1, 0, 2)
